# Initial kernel scaffold; baseline (speedup 1.0000x reference)
import jax
import jax.numpy as jnp
from jax.experimental import pallas as pl


def kernel(poses_3d, poses_2d_ref, vis_ref, cam_f, cam_c, num_persons_ref):
    raise NotImplementedError("write your pallas kernel here")


# speedup vs baseline: 6.7896x; 6.7896x over previous
import jax
import jax.numpy as jnp
from jax.experimental import pallas as pl
from jax.experimental.pallas import tpu as pltpu

_BONE_A = (0, 0, 1, 2, 5, 5, 7, 6, 8, 5, 6, 11, 11, 13, 12, 14)
_BONE_B = (1, 2, 3, 4, 6, 7, 9, 8, 10, 11, 12, 12, 13, 15, 14, 16)
_B, _NP, _NJ, _ND, _NB = 16, 20, 17, 64, 16
_IMG_W, _IMG_H = 1920.0, 1080.0
_SIGMA = 50.0


def _body(cams_ref, npr_ref, x3_ref, y3_ref, z3_ref, xr_ref, yr_ref, vis_ref,
          score_ref, sbl_ref, bound_ref, bound2_ref):
    b = pl.program_id(0)
    fx = cams_ref[b, 0]
    fy = cams_ref[b, 1]
    cx = cams_ref[b, 2]
    cy = cams_ref[b, 3]
    npr = npr_ref[b]

    z = jnp.maximum(z3_ref[0], 1e-3)
    xt = x3_ref[0] / z * fx + cx
    yt = y3_ref[0] / z * fy + cy
    xr = xr_ref[0]
    yr = yr_ref[0]
    vis = vis_ref[0]

    best_d = None
    best_i = None
    for pr in range(_NP):
        v = vis[pr][None, :, None]
        dx = xt - xr[pr][None, :, None]
        dy = yt - yr[pr][None, :, None]
        num = jnp.sum((dx * dx + dy * dy) * v, axis=1)
        den = jnp.sum(vis[pr]) + 1e-8
        d = num / den
        d = jnp.where(pr < npr, d, 1e5)
        if pr == 0:
            best_d = d
            best_i = jnp.zeros(d.shape, dtype=jnp.int32)
        else:
            take = d < best_d
            best_i = jnp.where(take, pr, best_i)
            best_d = jnp.where(take, d, best_d)

    mxr = jnp.zeros_like(xt)
    myr = jnp.zeros_like(yt)
    mvis = jnp.zeros_like(xt)
    for pr in range(_NP):
        m = (best_i == pr).astype(jnp.float32)[:, None, :]
        mxr = mxr + m * xr[pr][None, :, None]
        myr = myr + m * yr[pr][None, :, None]
        mvis = mvis + m * vis[pr][None, :, None]

    ddx = xt - mxr
    ddy = yt - myr
    md = ddx * ddx + ddy * ddy
    score_ref[0] = jnp.exp(-jnp.sqrt(md + 1e-12) / _SIGMA)

    def bone_len(px, py):
        outs = []
        for a, c in zip(_BONE_A, _BONE_B):
            ex = px[:, a] - px[:, c]
            ey = py[:, a] - py[:, c]
            outs.append(jnp.sqrt(ex * ex + ey * ey + 1e-12))
        return jnp.stack(outs, axis=1)

    bl_t = bone_len(xt, yt)
    bl_r = bone_len(mxr, myr)
    sbl_ref[0] = jnp.exp(-jnp.abs(bl_r - bl_t) / 5.0)

    inb = ((xt >= 0) & (yt >= 0) & (xt <= _IMG_W - 1) & (yt <= _IMG_H - 1))
    bound_ref[0] = inb.astype(jnp.float32) * mvis
    bound2_ref[0] = jnp.broadcast_to(mvis[:, :1, :], (_NP, _NB, _ND))


@jax.jit
def kernel(poses_3d, poses_2d_ref, vis_ref, cam_f, cam_c, num_persons_ref):
    x3 = poses_3d[..., 0]
    y3 = poses_3d[..., 1]
    z3 = poses_3d[..., 2]
    xr = poses_2d_ref[..., 0]
    yr = poses_2d_ref[..., 1]
    cams = jnp.concatenate([cam_f, cam_c], axis=1)

    full4 = pl.BlockSpec((1, _NP, _NJ, _ND), lambda b: (b, 0, 0, 0))
    full3 = pl.BlockSpec((1, _NP, _NJ), lambda b: (b, 0, 0))
    bone4 = pl.BlockSpec((1, _NP, _NB, _ND), lambda b: (b, 0, 0, 0))
    smem = pl.BlockSpec(memory_space=pltpu.SMEM)

    outs = pl.pallas_call(
        _body,
        grid=(_B,),
        in_specs=[smem, smem, full4, full4, full4, full3, full3, full3],
        out_specs=[full4, bone4, full4, bone4],
        out_shape=[
            jax.ShapeDtypeStruct((_B, _NP, _NJ, _ND), jnp.float32),
            jax.ShapeDtypeStruct((_B, _NP, _NB, _ND), jnp.float32),
            jax.ShapeDtypeStruct((_B, _NP, _NJ, _ND), jnp.float32),
            jax.ShapeDtypeStruct((_B, _NP, _NB, _ND), jnp.float32),
        ],
    )(cams, num_persons_ref, x3, y3, z3, xr, yr, vis_ref)
    return tuple(outs)
